# baseline (speedup 1.0000x reference)
import functools

import jax
import jax.numpy as jnp
import numpy as np
from jax import lax
from jax.experimental import pallas as pl
from jax.experimental.pallas import tpu as pltpu
from jax.experimental.pallas import tpu_sc as plsc

NUM_CORES = 2
NUM_SUBCORES = 16
NW = NUM_CORES * NUM_SUBCORES
LANES = 16
BATCH = 16384
FACTORS = 64
B_PER_W = BATCH // NW
NBLK = B_PER_W // LANES


def _body(users_hbm, items_hbm, gu_hbm, gi_hbm, bi_hbm,
          xui_out, beta_out, gu_out, gi_out,
          uidx_v, iidx_v, bufu, bufi, pu, pi, bv, xui_v, sem, semb):
  wid = lax.axis_index("s") * NUM_CORES + lax.axis_index("c")
  base = wid * B_PER_W

  pltpu.sync_copy(users_hbm.at[pl.ds(wid * NBLK, NBLK)], uidx_v)
  pltpu.sync_copy(items_hbm.at[pl.ds(wid * NBLK, NBLK)], iidx_v)

  bcopies = [
      pltpu.async_copy(bi_hbm.at[iidx_v.at[b]],
                       bv.at[pl.ds(b * LANES, LANES)], semb)
      for b in range(NBLK)
  ]
  for c in bcopies:
    c.wait()

  lane = lax.iota(jnp.int32, LANES)

  def block(b, _):
    uvec = uidx_v[b, pl.ds(0, LANES)]
    ivec = iidx_v[b, pl.ds(0, LANES)]
    for t in range(LANES):
      u = lax.squeeze(lax.slice(uvec, (t,), (t + 1,)), (0,))
      i = lax.squeeze(lax.slice(ivec, (t,), (t + 1,)), (0,))
      pltpu.async_copy(gu_hbm.at[pl.ds(u, 1)], bufu.at[pl.ds(t, 1)], sem)
      pltpu.async_copy(gi_hbm.at[pl.ds(i, 1)], bufi.at[pl.ds(t, 1)], sem)
    pltpu.make_async_copy(gu_hbm.at[pl.ds(0, LANES)], bufu, sem).wait()
    pltpu.make_async_copy(gi_hbm.at[pl.ds(0, LANES)], bufi, sem).wait()

    res = jnp.zeros((LANES,), jnp.float32)
    for t in range(LANES):
      prow = b * (LANES // 2) + t // 2
      pcol = (t % 2) * FACTORS
      vu = bufu[t, pl.ds(0, LANES)]
      vi = bufi[t, pl.ds(0, LANES)]
      pu[prow, pl.ds(pcol, LANES)] = vu
      pi[prow, pl.ds(pcol, LANES)] = vi
      acc = vu * vi
      for c in range(1, FACTORS // LANES):
        vu = bufu[t, pl.ds(c * LANES, LANES)]
        vi = bufi[t, pl.ds(c * LANES, LANES)]
        pu[prow, pl.ds(pcol + c * LANES, LANES)] = vu
        pi[prow, pl.ds(pcol + c * LANES, LANES)] = vi
        acc += vu * vi
      res = jnp.where(lane == t, jnp.sum(acc), res)
    xui_v[pl.ds(b * LANES, LANES)] = res + bv[pl.ds(b * LANES, LANES)]
    return 0

  lax.fori_loop(0, NBLK, block, 0)

  dst = pl.ds(wid * (B_PER_W // 2), B_PER_W // 2)
  pltpu.sync_copy(pu, gu_out.at[dst])
  pltpu.sync_copy(pi, gi_out.at[dst])
  pltpu.sync_copy(bv, beta_out.at[pl.ds(base, B_PER_W)])
  pltpu.sync_copy(xui_v, xui_out.at[pl.ds(base, B_PER_W)])


@jax.jit
def _run(users2, items2, Gu, Gi, bi_flat):
  mesh = plsc.VectorSubcoreMesh(core_axis_name="c", subcore_axis_name="s")
  f = pl.kernel(
      _body,
      out_type=(
          jax.ShapeDtypeStruct((BATCH,), jnp.float32),
          jax.ShapeDtypeStruct((BATCH,), jnp.float32),
          jax.ShapeDtypeStruct((BATCH // 2, 2 * FACTORS), jnp.float32),
          jax.ShapeDtypeStruct((BATCH // 2, 2 * FACTORS), jnp.float32),
      ),
      mesh=mesh,
      compiler_params=pltpu.CompilerParams(needs_layout_passes=False),
      scratch_types=[
          pltpu.VMEM((NBLK, LANES), jnp.int32),
          pltpu.VMEM((NBLK, LANES), jnp.int32),
          pltpu.VMEM((LANES, FACTORS), jnp.float32),
          pltpu.VMEM((LANES, FACTORS), jnp.float32),
          pltpu.VMEM((B_PER_W // 2, 2 * FACTORS), jnp.float32),
          pltpu.VMEM((B_PER_W // 2, 2 * FACTORS), jnp.float32),
          pltpu.VMEM((B_PER_W,), jnp.float32),
          pltpu.VMEM((B_PER_W,), jnp.float32),
          pltpu.SemaphoreType.DMA,
          pltpu.SemaphoreType.DMA,
      ],
  )
  return f(users2, items2, Gu, Gi, bi_flat)


def kernel(users_indices, items_indices, Gu, Gi, Bi):
  users2 = users_indices.astype(jnp.int32).reshape(BATCH // LANES, LANES)
  items2 = items_indices.astype(jnp.int32).reshape(BATCH // LANES, LANES)
  bi_flat = Bi.reshape(Bi.shape[0])
  xui, beta_i, gu2, gi2 = _run(users2, items2, Gu, Gi, bi_flat)
  gamma_u = gu2.reshape(BATCH, FACTORS)
  gamma_i = gi2.reshape(BATCH, FACTORS)
  return (xui, beta_i, gamma_u, gamma_i)
